# baseline (speedup 1.0000x reference)
import functools

import jax
import jax.numpy as jnp
from jax import lax
from jax.experimental import pallas as pl
from jax.experimental.pallas import tpu as pltpu
from jax.experimental.pallas import tpu_sc as plsc

B, S, E = 1024, 200, 128
V, K = 100000, 10
BV = 2048
NBV = 49
VPAD = NBV * BV
N = B * S * K

BH = B // 2
NHALF = N // 2
KS = K * S

NC, NS = 2, 16
NW = NC * NS
PER_W = NHALF // NW
CHUNK = 8000
NCHUNK = PER_W // CHUNK


def _matmul_body(tab_ref, ctx_ref, out_ref):
    s = lax.dot_general(
        tab_ref[...].astype(jnp.bfloat16), ctx_ref[...].astype(jnp.bfloat16),
        dimension_numbers=(((1,), (1,)), ((), ())),
        preferred_element_type=jnp.float32)
    sb = s.astype(jnp.bfloat16)
    lo = lax.bitcast_convert_type(sb[:BV // 2, :], jnp.int16)
    hi = lax.bitcast_convert_type(sb[BV // 2:, :], jnp.int16)
    w = (hi.astype(jnp.int32) << 16) | (lo.astype(jnp.int32) & 0xFFFF)
    out_ref[...] = w.reshape(BV // 2 * BH)


def _scores(ctx_half, emb_table):
    return pl.pallas_call(
        _matmul_body,
        grid=(NBV,),
        in_specs=[
            pl.BlockSpec((BV, E), lambda i: (i, 0)),
            pl.BlockSpec((BH, E), lambda i: (0, 0)),
        ],
        out_specs=pl.BlockSpec((BV // 2 * BH,), lambda i: (i,)),
        out_shape=jax.ShapeDtypeStruct((VPAD // 2 * BH,), jnp.int32),
    )(emb_table, ctx_half)


_sc_mesh = plsc.VectorSubcoreMesh(core_axis_name="c", subcore_axis_name="s")


@functools.partial(
    pl.kernel,
    mesh=_sc_mesh,
    out_type=jax.ShapeDtypeStruct((NHALF,), jnp.int32),
    scratch_types=[
        pltpu.VMEM((CHUNK,), jnp.int32),
        pltpu.VMEM((CHUNK,), jnp.int32),
        pltpu.VMEM((CHUNK,), jnp.int32),
        pltpu.VMEM((CHUNK,), jnp.int32),
        pltpu.SemaphoreType.DMA,
        pltpu.SemaphoreType.DMA,
        pltpu.SemaphoreType.DMA,
        pltpu.SemaphoreType.DMA,
        pltpu.SemaphoreType.DMA,
        pltpu.SemaphoreType.DMA,
    ],
)
def _sc_gather(scores_hbm, idx_hbm, out_hbm,
               i0, i1, v0, v1, si0, si1, sg0, sg1, so0, so1):
    idxb, valb = [i0, i1], [v0, v1]
    semi, semg, semo = [si0, si1], [sg0, sg1], [so0, so1]
    wid = lax.axis_index("s") * NC + lax.axis_index("c")
    base = wid * PER_W

    def start_idx(j):
        return pltpu.async_copy(
            idx_hbm.at[pl.ds(base + j * CHUNK, CHUNK)], idxb[j % 2],
            semi[j % 2])

    def start_gather(j):
        return pltpu.async_copy(scores_hbm.at[idxb[j % 2]], valb[j % 2],
                                semg[j % 2])

    def start_out(j):
        return pltpu.async_copy(
            valb[j % 2], out_hbm.at[pl.ds(base + j * CHUNK, CHUNK)],
            semo[j % 2])

    outs = [None, None]
    start_idx(0).wait()
    g = start_gather(0)
    for j in range(NCHUNK):
        if j + 1 < NCHUNK:
            start_idx(j + 1).wait()
            if j >= 1:
                outs[(j + 1) % 2].wait()
            g_next = start_gather(j + 1)
        g.wait()
        outs[j % 2] = start_out(j)
        if j + 1 < NCHUNK:
            g = g_next
    outs[(NCHUNK - 1) % 2].wait()
    outs[(NCHUNK - 2) % 2].wait()


BB = 64


def _pos_body(tgt_ref, ctx_ref, out_ref):
    i = pl.program_id(0)
    ctx = ctx_ref[...]
    x = tgt_ref[...] * ctx[:, None, :]
    pos = (jnp.minimum(x, 0.0) - jnp.log1p(jnp.exp(-jnp.abs(x)))).sum()

    @pl.when(i == 0)
    def _():
        out_ref[0, 0] = 0.0

    out_ref[0, 0] += -pos


def _pos_reduce(target_embs, context_tensor):
    out = pl.pallas_call(
        _pos_body,
        grid=(B // BB,),
        in_specs=[
            pl.BlockSpec((BB, S, E), lambda i: (i, 0, 0)),
            pl.BlockSpec((BB, E), lambda i: (i, 0)),
        ],
        out_specs=pl.BlockSpec(memory_space=pltpu.SMEM),
        out_shape=jax.ShapeDtypeStruct((1, 1), jnp.float32),
    )(target_embs, context_tensor)
    return out[0, 0]


NG = 2
GB = NHALF // NG


def _neg_body(g_ref, h_ref, out_ref):
    i = pl.program_id(0)
    w = g_ref[...].reshape(GB // 1024, 1024)
    h = h_ref[...].reshape(GB // 1024, 1024)
    bits = jnp.where(h == 1, w & jnp.int32(-65536), w << 16)
    vals = lax.bitcast_convert_type(bits, jnp.float32)
    neg = (jnp.minimum(-vals, 0.0)
           - jnp.log1p(jnp.exp(-jnp.abs(vals)))).sum()

    @pl.when(i == 0)
    def _():
        out_ref[0, 0] = 0.0

    out_ref[0, 0] += -neg


def _neg_reduce(gathered, halves):
    out = pl.pallas_call(
        _neg_body,
        grid=(NG,),
        in_specs=[pl.BlockSpec((GB,), lambda i: (i,)),
                  pl.BlockSpec((GB,), lambda i: (i,))],
        out_specs=pl.BlockSpec(memory_space=pltpu.SMEM),
        out_shape=jax.ShapeDtypeStruct((1, 1), jnp.float32),
    )(gathered, halves)
    return out[0, 0]


def kernel(target_embs, context_tensor, sample_ids, emb_table):
    ids3 = (sample_ids.astype(jnp.int32).transpose(2, 1, 0)
            .reshape(KS, 2, BH))
    r_idx = jnp.arange(BH, dtype=jnp.int32)[None, :]

    flat_idx, halves = [], []
    for h in range(2):
        idh = ids3[:, h, :]
        flat_idx.append(((idh >> 11) * (BV // 2 * BH)
                         + (idh & (BV // 2 - 1)) * BH + r_idx).reshape(NHALF))
        halves.append(((idh >> 10) & 1).reshape(NHALF))

    scores0 = _scores(lax.slice(context_tensor, (0, 0), (BH, E)), emb_table)
    gathered0 = _sc_gather(scores0, flat_idx[0])
    scores1 = _scores(lax.slice(context_tensor, (BH, 0), (B, E)), emb_table)
    gathered1 = _sc_gather(scores1, flat_idx[1])
    pos = _pos_reduce(target_embs, context_tensor)
    return (pos + _neg_reduce(gathered0, halves[0])
            + _neg_reduce(gathered1, halves[1]))


# speedup vs baseline: 1.2553x; 1.2553x over previous
import functools

import jax
import jax.numpy as jnp
from jax import lax
from jax.experimental import pallas as pl
from jax.experimental.pallas import tpu as pltpu
from jax.experimental.pallas import tpu_sc as plsc

B, S, E = 1024, 200, 128
V, K = 100000, 10
BV = 2048
NBV = 49
VPAD = NBV * BV
N = B * S * K

NC, NS = 2, 16
NW = NC * NS
PER_W = N // NW
CHUNK = 16000
NCHUNK = PER_W // CHUNK


def _matmul_body(tab_ref, ctx_ref, out_ref):
    s = lax.dot_general(
        tab_ref[...].astype(jnp.bfloat16), ctx_ref[...].astype(jnp.bfloat16),
        dimension_numbers=(((1,), (1,)), ((), ())),
        preferred_element_type=jnp.float32)
    sb = s.astype(jnp.bfloat16)
    lo = lax.bitcast_convert_type(sb[:BV // 2, :], jnp.int16)
    hi = lax.bitcast_convert_type(sb[BV // 2:, :], jnp.int16)
    w = (hi.astype(jnp.int32) << 16) | (lo.astype(jnp.int32) & 0xFFFF)
    out_ref[...] = w.reshape(BV // 2 * B)


def _scores(context_tensor, emb_table):
    return pl.pallas_call(
        _matmul_body,
        grid=(NBV,),
        in_specs=[
            pl.BlockSpec((BV, E), lambda i: (i, 0)),
            pl.BlockSpec((B, E), lambda i: (0, 0)),
        ],
        out_specs=pl.BlockSpec((BV // 2 * B,), lambda i: (i,)),
        out_shape=jax.ShapeDtypeStruct((VPAD // 2 * B,), jnp.int32),
    )(emb_table, context_tensor)


_sc_mesh = plsc.VectorSubcoreMesh(core_axis_name="c", subcore_axis_name="s")


@functools.partial(
    pl.kernel,
    mesh=_sc_mesh,
    out_type=jax.ShapeDtypeStruct((N,), jnp.int32),
    scratch_types=[
        pltpu.VMEM((CHUNK,), jnp.int32),
        pltpu.VMEM((CHUNK,), jnp.int32),
        pltpu.VMEM((CHUNK,), jnp.int32),
        pltpu.VMEM((CHUNK,), jnp.int32),
        pltpu.SemaphoreType.DMA,
        pltpu.SemaphoreType.DMA,
        pltpu.SemaphoreType.DMA,
        pltpu.SemaphoreType.DMA,
        pltpu.SemaphoreType.DMA,
        pltpu.SemaphoreType.DMA,
    ],
)
def _sc_gather(scores_hbm, idx_hbm, out_hbm,
               i0, i1, v0, v1, si0, si1, sg0, sg1, so0, so1):
    idxb, valb = [i0, i1], [v0, v1]
    semi, semg, semo = [si0, si1], [sg0, sg1], [so0, so1]
    wid = lax.axis_index("s") * NC + lax.axis_index("c")
    base = wid * PER_W

    def start_idx(j):
        return pltpu.async_copy(
            idx_hbm.at[pl.ds(base + j * CHUNK, CHUNK)], idxb[j % 2],
            semi[j % 2])

    def start_gather(j):
        return pltpu.async_copy(scores_hbm.at[idxb[j % 2]], valb[j % 2],
                                semg[j % 2])

    def start_out(j):
        return pltpu.async_copy(
            valb[j % 2], out_hbm.at[pl.ds(base + j * CHUNK, CHUNK)],
            semo[j % 2])

    outs = [None, None]
    start_idx(0).wait()
    g = start_gather(0)
    for j in range(NCHUNK):
        if j + 1 < NCHUNK:
            start_idx(j + 1).wait()
            if j >= 1:
                outs[(j + 1) % 2].wait()
            g_next = start_gather(j + 1)
        g.wait()
        outs[j % 2] = start_out(j)
        if j + 1 < NCHUNK:
            g = g_next
    outs[(NCHUNK - 1) % 2].wait()
    outs[(NCHUNK - 2) % 2].wait()


BB = 64


def _pos_body(tgt_ref, ctx_ref, out_ref):
    i = pl.program_id(0)
    ctx = ctx_ref[...]
    x = tgt_ref[...] * ctx[:, None, :]
    pos = (jnp.minimum(x, 0.0) - jnp.log1p(jnp.exp(-jnp.abs(x)))).sum()

    @pl.when(i == 0)
    def _():
        out_ref[0, 0] = 0.0

    out_ref[0, 0] += -pos


def _pos_reduce(target_embs, context_tensor):
    out = pl.pallas_call(
        _pos_body,
        grid=(B // BB,),
        in_specs=[
            pl.BlockSpec((BB, S, E), lambda i: (i, 0, 0)),
            pl.BlockSpec((BB, E), lambda i: (i, 0)),
        ],
        out_specs=pl.BlockSpec(memory_space=pltpu.SMEM),
        out_shape=jax.ShapeDtypeStruct((1, 1), jnp.float32),
    )(target_embs, context_tensor)
    return out[0, 0]


NG = 4
GB = N // NG


def _neg_body(g_ref, h_ref, out_ref):
    i = pl.program_id(0)
    w = g_ref[...].reshape(GB // 1024, 1024)
    h = h_ref[...].reshape(GB // 1024, 1024)
    bits = jnp.where(h == 1, w & jnp.int32(-65536), w << 16)
    vals = lax.bitcast_convert_type(bits, jnp.float32)
    neg = (jnp.minimum(-vals, 0.0)
           - jnp.log1p(jnp.exp(-jnp.abs(vals)))).sum()

    @pl.when(i == 0)
    def _():
        out_ref[0, 0] = 0.0

    out_ref[0, 0] += -neg


def _neg_reduce(gathered, halves):
    out = pl.pallas_call(
        _neg_body,
        grid=(NG,),
        in_specs=[pl.BlockSpec((GB,), lambda i: (i,)),
                  pl.BlockSpec((GB,), lambda i: (i,))],
        out_specs=pl.BlockSpec(memory_space=pltpu.SMEM),
        out_shape=jax.ShapeDtypeStruct((1, 1), jnp.float32),
    )(gathered, halves)
    return out[0, 0]


def kernel(target_embs, context_tensor, sample_ids, emb_table):
    scores = _scores(context_tensor, emb_table)
    ids = sample_ids.astype(jnp.int32).transpose(2, 1, 0).reshape(N)
    b_idx = jnp.arange(N, dtype=jnp.int32) & (B - 1)
    flat_idx = ((ids >> 11) * (BV // 2 * B) + (ids & (BV // 2 - 1)) * B
                + b_idx)
    halves = (ids >> 10) & 1
    gathered = _sc_gather(scores, flat_idx)
    return _pos_reduce(target_embs, context_tensor) + _neg_reduce(gathered,
                                                                  halves)


# speedup vs baseline: 1.2792x; 1.0191x over previous
import functools

import jax
import jax.numpy as jnp
from jax import lax
from jax.experimental import pallas as pl
from jax.experimental.pallas import tpu as pltpu
from jax.experimental.pallas import tpu_sc as plsc

B, S, E = 1024, 200, 128
V, K = 100000, 10
BV = 4096
NBV = 25
VPAD = NBV * BV
SH1 = BV.bit_length() - 1
SH2 = SH1 - 1
N = B * S * K

NC, NS = 2, 16
NW = NC * NS
PER_W = N // NW
CHUNK = 16000
NCHUNK = PER_W // CHUNK


def _matmul_body(tab_ref, ctx_ref, out_ref):
    s = lax.dot_general(
        tab_ref[...].astype(jnp.bfloat16), ctx_ref[...].astype(jnp.bfloat16),
        dimension_numbers=(((1,), (1,)), ((), ())),
        preferred_element_type=jnp.float32)
    sb = s.astype(jnp.bfloat16)
    lo = lax.bitcast_convert_type(sb[:BV // 2, :], jnp.int16)
    hi = lax.bitcast_convert_type(sb[BV // 2:, :], jnp.int16)
    w = (hi.astype(jnp.int32) << 16) | (lo.astype(jnp.int32) & 0xFFFF)
    out_ref[...] = w.reshape(BV // 2 * B)


def _scores(context_tensor, emb_table):
    return pl.pallas_call(
        _matmul_body,
        grid=(NBV,),
        in_specs=[
            pl.BlockSpec((BV, E), lambda i: (i, 0)),
            pl.BlockSpec((B, E), lambda i: (0, 0)),
        ],
        out_specs=pl.BlockSpec((BV // 2 * B,), lambda i: (i,)),
        out_shape=jax.ShapeDtypeStruct((VPAD // 2 * B,), jnp.int32),
    )(emb_table, context_tensor)


_sc_mesh = plsc.VectorSubcoreMesh(core_axis_name="c", subcore_axis_name="s")


@functools.partial(
    pl.kernel,
    mesh=_sc_mesh,
    out_type=jax.ShapeDtypeStruct((N,), jnp.int32),
    scratch_types=[
        pltpu.VMEM((CHUNK,), jnp.int32),
        pltpu.VMEM((CHUNK,), jnp.int32),
        pltpu.VMEM((CHUNK,), jnp.int32),
        pltpu.VMEM((CHUNK,), jnp.int32),
        pltpu.SemaphoreType.DMA,
        pltpu.SemaphoreType.DMA,
        pltpu.SemaphoreType.DMA,
        pltpu.SemaphoreType.DMA,
        pltpu.SemaphoreType.DMA,
        pltpu.SemaphoreType.DMA,
    ],
)
def _sc_gather(scores_hbm, idx_hbm, out_hbm,
               i0, i1, v0, v1, si0, si1, sg0, sg1, so0, so1):
    idxb, valb = [i0, i1], [v0, v1]
    semi, semg, semo = [si0, si1], [sg0, sg1], [so0, so1]
    wid = lax.axis_index("s") * NC + lax.axis_index("c")
    base = wid * PER_W

    def start_idx(j):
        return pltpu.async_copy(
            idx_hbm.at[pl.ds(base + j * CHUNK, CHUNK)], idxb[j % 2],
            semi[j % 2])

    def start_gather(j):
        return pltpu.async_copy(scores_hbm.at[idxb[j % 2]], valb[j % 2],
                                semg[j % 2])

    def start_out(j):
        return pltpu.async_copy(
            valb[j % 2], out_hbm.at[pl.ds(base + j * CHUNK, CHUNK)],
            semo[j % 2])

    outs = [None, None]
    start_idx(0).wait()
    g = start_gather(0)
    for j in range(NCHUNK):
        if j + 1 < NCHUNK:
            start_idx(j + 1).wait()
            if j >= 1:
                outs[(j + 1) % 2].wait()
            g_next = start_gather(j + 1)
        g.wait()
        outs[j % 2] = start_out(j)
        if j + 1 < NCHUNK:
            g = g_next
    outs[(NCHUNK - 1) % 2].wait()
    outs[(NCHUNK - 2) % 2].wait()


BB = 64


def _pos_body(tgt_ref, ctx_ref, out_ref):
    i = pl.program_id(0)
    ctx = ctx_ref[...]
    x = tgt_ref[...] * ctx[:, None, :]
    pos = (jnp.minimum(x, 0.0) - jnp.log1p(jnp.exp(-jnp.abs(x)))).sum()

    @pl.when(i == 0)
    def _():
        out_ref[0, 0] = 0.0

    out_ref[0, 0] += -pos


def _pos_reduce(target_embs, context_tensor):
    out = pl.pallas_call(
        _pos_body,
        grid=(B // BB,),
        in_specs=[
            pl.BlockSpec((BB, S, E), lambda i: (i, 0, 0)),
            pl.BlockSpec((BB, E), lambda i: (i, 0)),
        ],
        out_specs=pl.BlockSpec(memory_space=pltpu.SMEM),
        out_shape=jax.ShapeDtypeStruct((1, 1), jnp.float32),
    )(target_embs, context_tensor)
    return out[0, 0]


NG = 4
GB = N // NG


def _neg_body(g_ref, h_ref, out_ref):
    i = pl.program_id(0)
    w = g_ref[...].reshape(GB // 1024, 1024)
    h = h_ref[...].reshape(GB // 1024, 1024)
    bits = jnp.where(h == 1, w & jnp.int32(-65536), w << 16)
    vals = lax.bitcast_convert_type(bits, jnp.float32)
    neg = (jnp.minimum(-vals, 0.0)
           - jnp.log1p(jnp.exp(-jnp.abs(vals)))).sum()

    @pl.when(i == 0)
    def _():
        out_ref[0, 0] = 0.0

    out_ref[0, 0] += -neg


def _neg_reduce(gathered, halves):
    out = pl.pallas_call(
        _neg_body,
        grid=(NG,),
        in_specs=[pl.BlockSpec((GB,), lambda i: (i,)),
                  pl.BlockSpec((GB,), lambda i: (i,))],
        out_specs=pl.BlockSpec(memory_space=pltpu.SMEM),
        out_shape=jax.ShapeDtypeStruct((1, 1), jnp.float32),
    )(gathered, halves)
    return out[0, 0]


def kernel(target_embs, context_tensor, sample_ids, emb_table):
    scores = _scores(context_tensor, emb_table)
    ids = sample_ids.astype(jnp.int32).transpose(2, 1, 0).reshape(N)
    b_idx = jnp.arange(N, dtype=jnp.int32) & (B - 1)
    flat_idx = ((ids >> SH1) * (BV // 2 * B) + (ids & (BV // 2 - 1)) * B
                + b_idx)
    halves = (ids >> SH2) & 1
    gathered = _sc_gather(scores, flat_idx)
    return _pos_reduce(target_embs, context_tensor) + _neg_reduce(gathered,
                                                                  halves)
